# Initial kernel scaffold; baseline (speedup 1.0000x reference)
import functools

import jax
import jax.numpy as jnp
from jax import lax
from jax.experimental import pallas as pl
from jax.experimental.pallas import tpu as pltpu
from jax.experimental.pallas import tpu_sc as plsc

_N = 10000
_E = 320000
_D = 128
_H = 128
_C = 40

_NC = 2
_NS = 16
_K = 80
_EPS = _E // _NS
_NCHUNK = _EPS // _K
_RPS = _N // _NS
_WB = 125
_NWB = _RPS // _WB

_sc_mesh = plsc.VectorSubcoreMesh(core_axis_name="c", subcore_axis_name="s")


def _make_prop(width, with_deg):
  out_type = [jax.ShapeDtypeStruct((_NC, _N, width), jnp.float32)]
  scratch = dict(
      src_v=pltpu.VMEM((_NCHUNK, _K), jnp.int32),
      dst_v=pltpu.VMEM((_NCHUNK, _K), jnp.int32),
      rows_v=pltpu.VMEM((_K, width), jnp.float32),
      wb_v=pltpu.VMEM((_WB, width), jnp.float32),
      acc=pltpu.VMEM_SHARED((_N, width), jnp.float32),
      sem=pltpu.SemaphoreType.DMA,
  )
  if with_deg:
    out_type.append(jax.ShapeDtypeStruct((_NC, _N, 16), jnp.float32))
    scratch.update(
        ones_v=pltpu.VMEM((_K, 16), jnp.float32),
        dwb_v=pltpu.VMEM((_RPS, 16), jnp.float32),
        dacc=pltpu.VMEM_SHARED((_N, 16), jnp.float32),
    )

  def body(h_hbm, adj_hbm, zeros_hbm, *rest):
    if with_deg:
      (ones_hbm, out_hbm, outd_hbm, src_v, dst_v, rows_v, wb_v, acc, sem,
       ones_v, dwb_v, dacc) = rest
    else:
      out_hbm, src_v, dst_v, rows_v, wb_v, acc, sem = rest
    c = lax.axis_index("c")
    s = lax.axis_index("s")
    r0 = s * _RPS

    pltpu.sync_copy(adj_hbm.at[c, 0, s], src_v)
    pltpu.sync_copy(adj_hbm.at[c, 1, s], dst_v)

    pltpu.sync_copy(zeros_hbm, wb_v)
    for w in range(_NWB):
      pltpu.sync_copy(wb_v, acc.at[pl.ds(r0 + w * _WB, _WB)])
    if with_deg:
      pltpu.sync_copy(ones_hbm, ones_v)
      pltpu.sync_copy(zeros_hbm.at[:, pl.ds(0, 16)], dwb_v.at[pl.ds(0, _WB)])
      for w in range(_NWB):
        pltpu.sync_copy(dwb_v.at[pl.ds(0, _WB)],
                        dacc.at[pl.ds(r0 + w * _WB, _WB)])
    plsc.subcore_barrier()

    def step(j, carry):
      pltpu.async_copy(h_hbm.at[src_v.at[j]], rows_v, sem).wait()
      pltpu.sync_copy(rows_v, acc.at[dst_v.at[j]], add=True)
      if with_deg:
        pltpu.sync_copy(ones_v, dacc.at[dst_v.at[j]], add=True)
      return carry

    lax.fori_loop(0, _NCHUNK, step, 0)
    plsc.subcore_barrier()

    for w in range(_NWB):
      pltpu.sync_copy(acc.at[pl.ds(r0 + w * _WB, _WB)], wb_v)
      pltpu.sync_copy(wb_v, out_hbm.at[c, pl.ds(r0 + w * _WB, _WB)])
    if with_deg:
      pltpu.sync_copy(dacc.at[pl.ds(r0, _RPS)], dwb_v)
      pltpu.sync_copy(dwb_v, outd_hbm.at[c, pl.ds(r0, _RPS)])

  return pl.kernel(body, out_type=out_type, mesh=_sc_mesh,
                   scratch_types=scratch)


_prop128_deg = _make_prop(128, True)
_prop128 = _make_prop(128, False)
_prop64 = _make_prop(64, False)


_BM = 1250
_GRID = _N // _BM


def _fc1_body(x_ref, w_ref, b_ref, o_ref):
  o_ref[...] = jnp.dot(x_ref[...], w_ref[...],
                       preferred_element_type=jnp.float32,
                       precision=lax.Precision.HIGHEST) + b_ref[...]


def _fc1(x, w, b):
  return pl.pallas_call(
      _fc1_body,
      grid=(_GRID,),
      in_specs=[
          pl.BlockSpec((_BM, _D), lambda i: (i, 0)),
          pl.BlockSpec((_D, _H), lambda i: (0, 0)),
          pl.BlockSpec((1, _H), lambda i: (0, 0)),
      ],
      out_specs=pl.BlockSpec((_BM, _H), lambda i: (i, 0)),
      out_shape=jax.ShapeDtypeStruct((_N, _H), jnp.float32),
  )(x, w, b)


def _layer_body(s_ref, inv_ref, h_ref, w_ref, b_ref, o_ref):
  agg = 0.5 * (s_ref[0] * inv_ref[0] + s_ref[1] * inv_ref[1])
  out = jnp.dot(agg, w_ref[...], preferred_element_type=jnp.float32,
                precision=lax.Precision.HIGHEST) + b_ref[...]
  o_ref[...] = jnp.maximum(out + h_ref[...], 0.0)


def _layer(s, inv, h, w, b):
  return pl.pallas_call(
      _layer_body,
      grid=(_GRID,),
      in_specs=[
          pl.BlockSpec((2, _BM, _H), lambda i: (0, i, 0)),
          pl.BlockSpec((2, _BM, 1), lambda i: (0, i, 0)),
          pl.BlockSpec((_BM, _H), lambda i: (i, 0)),
          pl.BlockSpec((_H, _H), lambda i: (0, 0)),
          pl.BlockSpec((1, _H), lambda i: (0, 0)),
      ],
      out_specs=pl.BlockSpec((_BM, _H), lambda i: (i, 0)),
      out_shape=jax.ShapeDtypeStruct((_N, _H), jnp.float32),
  )(s, inv, h, w, b)


def _mm_body(x_ref, w_ref, o_ref):
  o_ref[...] = jnp.dot(x_ref[...], w_ref[...],
                       preferred_element_type=jnp.float32,
                       precision=lax.Precision.HIGHEST)


def _mm64(x, w):
  return pl.pallas_call(
      _mm_body,
      grid=(_GRID,),
      in_specs=[
          pl.BlockSpec((_BM, _H), lambda i: (i, 0)),
          pl.BlockSpec((_H, 64), lambda i: (0, 0)),
      ],
      out_specs=pl.BlockSpec((_BM, 64), lambda i: (i, 0)),
      out_shape=jax.ShapeDtypeStruct((_N, 64), jnp.float32),
  )(x, w)


def _final_body(t_ref, inv_ref, b_ref, o_ref):
  agg = 0.5 * (t_ref[0] * inv_ref[0] + t_ref[1] * inv_ref[1])
  o_ref[...] = jnp.maximum(agg + b_ref[...], 0.0)


def _final(t, inv, b):
  return pl.pallas_call(
      _final_body,
      grid=(_GRID,),
      in_specs=[
          pl.BlockSpec((2, _BM, 64), lambda i: (0, i, 0)),
          pl.BlockSpec((2, _BM, 1), lambda i: (0, i, 0)),
          pl.BlockSpec((1, 64), lambda i: (0, 0)),
      ],
      out_specs=pl.BlockSpec((_BM, 64), lambda i: (i, 0)),
      out_shape=jax.ShapeDtypeStruct((_N, 64), jnp.float32),
  )(t, inv, b)


def kernel(x, adj1, adj2, W_fc1, b_fc1, W0, b0, W1, b1, W2, b2):
  adj = jnp.stack([adj1, adj2]).reshape(_NC, 2, _NS, _NCHUNK, _K)
  zeros = jnp.zeros((_WB, _H), jnp.float32)
  ones16 = jnp.ones((_K, 16), jnp.float32)

  h = _fc1(x, W_fc1, b_fc1.reshape(1, _H))

  s, degs = _prop128_deg(h, adj, zeros, ones16)
  inv = 1.0 / jnp.clip(degs[:, :, :1], 1.0, None)

  h = _layer(s, inv, h, W0, b0.reshape(1, _H))
  s = _prop128(h, adj, zeros)
  h = _layer(s, inv, h, W1, b1.reshape(1, _H))

  w2p = jnp.pad(W2, ((0, 0), (0, 64 - _C)))
  b2p = jnp.pad(b2, (0, 64 - _C)).reshape(1, 64)
  g = _mm64(h, w2p)
  t = _prop64(g, adj, zeros[:, :64])
  out = _final(t, inv, b2p)
  return out[:, :_C]


# speedup vs baseline: 2.1291x; 2.1291x over previous
import jax
import jax.numpy as jnp
from jax import lax
from jax.experimental import pallas as pl
from jax.experimental.pallas import tpu as pltpu
from jax.experimental.pallas import tpu_sc as plsc

_N = 10000
_E = 320000
_D = 128
_H = 128
_C = 40

_NC = 2
_NS = 16
_K = 128
_BLK = 4
_NBLK = 40
_EPS = _E // _NS
_EPP = _NBLK * _BLK * _K
_NP = 10240
_RPS = _NP // _NS
_CW = 128
_NWB = _RPS // _CW

_sc_mesh = plsc.VectorSubcoreMesh(core_axis_name="c", subcore_axis_name="s")


def _make_prop(with_deg, edge_mode=2):
  out_type = [jax.ShapeDtypeStruct((_NC, _NP, _H), jnp.float32)]
  scratch = [
      pltpu.VMEM((_BLK, _K), jnp.int32),
      pltpu.VMEM((_BLK, _K), jnp.int32),
      pltpu.VMEM((_K, _H), jnp.float32),
      pltpu.VMEM_SHARED((_NP, _H), jnp.float32),
      pltpu.SemaphoreType.DMA,
  ]
  if with_deg:
    out_type.append(jax.ShapeDtypeStruct((_NC, _NP, 16), jnp.float32))
    scratch += [
        pltpu.VMEM((_CW, 16), jnp.float32),
        pltpu.VMEM_SHARED((_NP, 16), jnp.float32),
    ]

  def body(h_hbm, adj_hbm, zeros_hbm, *rest):
    if with_deg:
      (zeros16_hbm, out_hbm, outd_hbm, src_v, dst_v, rows_v,
       acc, sem, dwb_v, dacc) = rest
    else:
      out_hbm, src_v, dst_v, rows_v, acc, sem = rest
    c = lax.axis_index("c")
    s = lax.axis_index("s")
    r0 = s * _RPS

    pltpu.sync_copy(zeros_hbm, rows_v)
    if with_deg:
      pltpu.sync_copy(zeros16_hbm, dwb_v)
    for w in range(_NWB):
      o = pl.multiple_of(r0 + w * _CW, 8)
      pltpu.sync_copy(rows_v, acc.at[pl.ds(o, _CW)])
      if with_deg:
        pltpu.sync_copy(dwb_v, dacc.at[pl.ds(o, _CW)])
    if with_deg:
      for i in range(_K):
        dwb_v[i, :] = jnp.ones((16,), jnp.float32)
    plsc.subcore_barrier()

    if edge_mode >= 1:
      @pl.loop(0, _NBLK)
      def _block(b):
        pltpu.sync_copy(adj_hbm.at[c, 0, s, b], src_v)
        pltpu.sync_copy(adj_hbm.at[c, 1, s, b], dst_v)
        for j in range(_BLK):
          pltpu.async_copy(h_hbm.at[src_v.at[j]], rows_v, sem).wait()
          if edge_mode >= 2:
            pltpu.sync_copy(rows_v, acc.at[dst_v.at[j]], add=True)
            if with_deg:
              pltpu.sync_copy(dwb_v, dacc.at[dst_v.at[j]], add=True)

    plsc.subcore_barrier()

    for w in range(_NWB):
      o = pl.multiple_of(r0 + w * _CW, 8)
      pltpu.sync_copy(acc.at[pl.ds(o, _CW)], rows_v)
      pltpu.sync_copy(rows_v, out_hbm.at[c, pl.ds(o, _CW)])
      if with_deg:
        pltpu.sync_copy(dacc.at[pl.ds(o, _CW)], dwb_v)
        pltpu.sync_copy(dwb_v, outd_hbm.at[c, pl.ds(o, _CW)])

  if not with_deg:
    out_type = out_type[0]
  return pl.kernel(body, out_type=out_type, mesh=_sc_mesh,
                   scratch_types=scratch)


_prop_deg = _make_prop(True)
_prop = _make_prop(False)


_BM = 2000
_GRID = _N // _BM


def _fc1_body(x_ref, w_ref, b_ref, o_ref):
  o_ref[...] = jnp.dot(x_ref[...], w_ref[...],
                       preferred_element_type=jnp.float32,
                       precision=lax.Precision.HIGHEST) + b_ref[...]


def _fc1(x, w, b):
  return pl.pallas_call(
      _fc1_body,
      grid=(_GRID,),
      in_specs=[
          pl.BlockSpec((_BM, _D), lambda i: (i, 0)),
          pl.BlockSpec((_D, _H), lambda i: (0, 0)),
          pl.BlockSpec((1, _H), lambda i: (0, 0)),
      ],
      out_specs=pl.BlockSpec((_BM, _H), lambda i: (i, 0)),
      out_shape=jax.ShapeDtypeStruct((_N, _H), jnp.float32),
  )(x, w, b)


def _layer_body(s_ref, inv_ref, h_ref, w_ref, b_ref, o_ref):
  agg = 0.5 * (s_ref[0] * inv_ref[0] + s_ref[1] * inv_ref[1])
  out = jnp.dot(agg, w_ref[...], preferred_element_type=jnp.float32,
                precision=lax.Precision.HIGHEST) + b_ref[...]
  o_ref[...] = jnp.maximum(out + h_ref[...], 0.0)


def _layer(s, inv, h, w, b):
  return pl.pallas_call(
      _layer_body,
      grid=(_GRID,),
      in_specs=[
          pl.BlockSpec((2, _BM, _H), lambda i: (0, i, 0)),
          pl.BlockSpec((2, _BM, 1), lambda i: (0, i, 0)),
          pl.BlockSpec((_BM, _H), lambda i: (i, 0)),
          pl.BlockSpec((_H, _H), lambda i: (0, 0)),
          pl.BlockSpec((1, _H), lambda i: (0, 0)),
      ],
      out_specs=pl.BlockSpec((_BM, _H), lambda i: (i, 0)),
      out_shape=jax.ShapeDtypeStruct((_N, _H), jnp.float32),
  )(s, inv, h, w, b)


def _layer2_body(s_ref, inv_ref, w_ref, b_ref, o_ref):
  agg = 0.5 * (s_ref[0] * inv_ref[0] + s_ref[1] * inv_ref[1])
  out = jnp.dot(agg, w_ref[...], preferred_element_type=jnp.float32,
                precision=lax.Precision.HIGHEST) + b_ref[...]
  o_ref[...] = jnp.maximum(out, 0.0)


def _layer2(s, inv, w, b):
  return pl.pallas_call(
      _layer2_body,
      grid=(_GRID,),
      in_specs=[
          pl.BlockSpec((2, _BM, _H), lambda i: (0, i, 0)),
          pl.BlockSpec((2, _BM, 1), lambda i: (0, i, 0)),
          pl.BlockSpec((_H, 64), lambda i: (0, 0)),
          pl.BlockSpec((1, 64), lambda i: (0, 0)),
      ],
      out_specs=pl.BlockSpec((_BM, 64), lambda i: (i, 0)),
      out_shape=jax.ShapeDtypeStruct((_N, 64), jnp.float32),
  )(s, inv, w, b)


def _pad_adj(a):
  src = jnp.pad(a[0].reshape(_NS, _EPS), ((0, 0), (0, _EPP - _EPS)))
  dst = jnp.pad(a[1].reshape(_NS, _EPS), ((0, 0), (0, _EPP - _EPS)),
                constant_values=_N)
  return jnp.stack([src, dst]).reshape(2, _NS, _NBLK, _BLK, _K)


def kernel(x, adj1, adj2, W_fc1, b_fc1, W0, b0, W1, b1, W2, b2):
  adj = jnp.stack([_pad_adj(adj1), _pad_adj(adj2)])
  zeros = jnp.zeros((_CW, _H), jnp.float32)
  zeros16 = jnp.zeros((_CW, 16), jnp.float32)

  h = _fc1(x, W_fc1, b_fc1.reshape(1, _H))

  degs = _prop(jnp.ones((_N, _H), jnp.float32), adj, zeros)
  inv = 1.0 / jnp.clip(degs[:, :_N, :1], 1.0, None)

  s = _prop(h, adj, zeros)
  h = _layer(s, inv, h, W0, b0.reshape(1, _H))
  s = _prop(h, adj, zeros)
  h = _layer(s, inv, h, W1, b1.reshape(1, _H))

  w2p = jnp.pad(W2, ((0, 0), (0, 64 - _C)))
  b2p = jnp.pad(b2, (0, 64 - _C)).reshape(1, 64)
  s = _prop(h, adj, zeros)
  out = _layer2(s, inv, w2p, b2p)
  return out[:, :_C]


# speedup vs baseline: 2.4393x; 1.1457x over previous
import jax
import jax.numpy as jnp
from jax import lax
from jax.experimental import pallas as pl
from jax.experimental.pallas import tpu as pltpu
from jax.experimental.pallas import tpu_sc as plsc

_N = 10000
_E = 320000
_D = 128
_H = 128
_C = 40

_NC = 2
_NS = 16
_K = 128
_BLK = 4
_NBLK = 40
_EPS = _E // _NS
_EPP = _NBLK * _BLK * _K
_NP = 10240
_RPS = _NP // _NS
_CW = 128
_NWB = _RPS // _CW

_sc_mesh = plsc.VectorSubcoreMesh(core_axis_name="c", subcore_axis_name="s")


def _make_prop():
  scratch = [
      pltpu.VMEM((_BLK, _K), jnp.int32),
      pltpu.VMEM((_BLK, _K), jnp.int32),
      pltpu.VMEM((_BLK, _K), jnp.int32),
      pltpu.VMEM((_BLK, _K), jnp.int32),
      pltpu.VMEM((_K, _H), jnp.float32),
      pltpu.VMEM((_K, _H), jnp.float32),
      pltpu.SemaphoreType.DMA,
      pltpu.SemaphoreType.DMA,
      pltpu.SemaphoreType.DMA,
      pltpu.SemaphoreType.DMA,
      pltpu.VMEM_SHARED((_NP, _H), jnp.float32),
  ]

  def body(h_hbm, adj_hbm, zeros_hbm, out_hbm,
           src0, dst0, src1, dst1, rows0, rows1,
           sem_g0, sem_g1, sem_is, sem_id, acc):
    c = lax.axis_index("c")
    s = lax.axis_index("s")
    r0 = s * _RPS
    srcs, dsts = (src0, src1), (dst0, dst1)
    rows, sem_g = (rows0, rows1), (sem_g0, sem_g1)

    pltpu.sync_copy(zeros_hbm, rows0)
    for w in range(_NWB):
      o = pl.multiple_of(r0 + w * _CW, 8)
      pltpu.sync_copy(rows0, acc.at[pl.ds(o, _CW)])
    plsc.subcore_barrier()

    pltpu.sync_copy(adj_hbm.at[c, 0, s, 0], src0)
    pltpu.sync_copy(adj_hbm.at[c, 1, s, 0], dst0)
    pltpu.async_copy(h_hbm.at[src0.at[0]], rows0, sem_g0)

    @pl.loop(0, _NBLK, step=2)
    def _pair(b):
      for pb in range(2):
        bb = b + pb
        sv, dv = srcs[pb], dsts[pb]
        nsv, ndv = srcs[pb ^ 1], dsts[pb ^ 1]
        for j in range(_BLK):
          par = j % 2
          pltpu.make_async_copy(h_hbm.at[sv.at[j]], rows[par],
                                sem_g[par]).wait()
          if j == 0:
            @pl.when(bb + 1 < _NBLK)
            def _():
              pltpu.async_copy(adj_hbm.at[c, 0, s, bb + 1], nsv, sem_is)
              pltpu.async_copy(adj_hbm.at[c, 1, s, bb + 1], ndv, sem_id)
          if j < _BLK - 1:
            pltpu.async_copy(h_hbm.at[sv.at[j + 1]], rows[par ^ 1],
                             sem_g[par ^ 1])
          else:
            @pl.when(bb + 1 < _NBLK)
            def _():
              pltpu.make_async_copy(adj_hbm.at[c, 0, s, bb + 1], nsv,
                                    sem_is).wait()
              pltpu.make_async_copy(adj_hbm.at[c, 1, s, bb + 1], ndv,
                                    sem_id).wait()
              pltpu.async_copy(h_hbm.at[nsv.at[0]], rows[par ^ 1],
                               sem_g[par ^ 1])
          pltpu.sync_copy(rows[par], acc.at[dv.at[j]], add=True)

    plsc.subcore_barrier()

    for w in range(_NWB):
      o = pl.multiple_of(r0 + w * _CW, 8)
      pltpu.sync_copy(acc.at[pl.ds(o, _CW)], rows0)
      pltpu.sync_copy(rows0, out_hbm.at[c, pl.ds(o, _CW)])

  return pl.kernel(body,
                   out_type=jax.ShapeDtypeStruct((_NC, _NP, _H), jnp.float32),
                   mesh=_sc_mesh, scratch_types=scratch)


_prop = _make_prop()


_BM = 2000
_GRID = _N // _BM


def _fc1_body(x_ref, w_ref, b_ref, o_ref):
  o_ref[...] = jnp.dot(x_ref[...], w_ref[...],
                       preferred_element_type=jnp.float32,
                       precision=lax.Precision.HIGHEST) + b_ref[...]


def _fc1(x, w, b):
  return pl.pallas_call(
      _fc1_body,
      grid=(_GRID,),
      in_specs=[
          pl.BlockSpec((_BM, _D), lambda i: (i, 0)),
          pl.BlockSpec((_D, _H), lambda i: (0, 0)),
          pl.BlockSpec((1, _H), lambda i: (0, 0)),
      ],
      out_specs=pl.BlockSpec((_BM, _H), lambda i: (i, 0)),
      out_shape=jax.ShapeDtypeStruct((_N, _H), jnp.float32),
  )(x, w, b)


def _layer_body(s_ref, inv_ref, h_ref, w_ref, b_ref, o_ref):
  agg = 0.5 * (s_ref[0] * inv_ref[0] + s_ref[1] * inv_ref[1])
  out = jnp.dot(agg, w_ref[...], preferred_element_type=jnp.float32,
                precision=lax.Precision.HIGHEST) + b_ref[...]
  o_ref[...] = jnp.maximum(out + h_ref[...], 0.0)


def _layer(s, inv, h, w, b):
  return pl.pallas_call(
      _layer_body,
      grid=(_GRID,),
      in_specs=[
          pl.BlockSpec((2, _BM, _H), lambda i: (0, i, 0)),
          pl.BlockSpec((2, _BM, 1), lambda i: (0, i, 0)),
          pl.BlockSpec((_BM, _H), lambda i: (i, 0)),
          pl.BlockSpec((_H, _H), lambda i: (0, 0)),
          pl.BlockSpec((1, _H), lambda i: (0, 0)),
      ],
      out_specs=pl.BlockSpec((_BM, _H), lambda i: (i, 0)),
      out_shape=jax.ShapeDtypeStruct((_N, _H), jnp.float32),
  )(s, inv, h, w, b)


def _layer2_body(s_ref, inv_ref, w_ref, b_ref, o_ref):
  agg = 0.5 * (s_ref[0] * inv_ref[0] + s_ref[1] * inv_ref[1])
  out = jnp.dot(agg, w_ref[...], preferred_element_type=jnp.float32,
                precision=lax.Precision.HIGHEST) + b_ref[...]
  o_ref[...] = jnp.maximum(out, 0.0)


def _layer2(s, inv, w, b):
  return pl.pallas_call(
      _layer2_body,
      grid=(_GRID,),
      in_specs=[
          pl.BlockSpec((2, _BM, _H), lambda i: (0, i, 0)),
          pl.BlockSpec((2, _BM, 1), lambda i: (0, i, 0)),
          pl.BlockSpec((_H, 64), lambda i: (0, 0)),
          pl.BlockSpec((1, 64), lambda i: (0, 0)),
      ],
      out_specs=pl.BlockSpec((_BM, 64), lambda i: (i, 0)),
      out_shape=jax.ShapeDtypeStruct((_N, 64), jnp.float32),
  )(s, inv, w, b)


def _pad_adj(a):
  src = jnp.pad(a[0].reshape(_NS, _EPS), ((0, 0), (0, _EPP - _EPS)))
  dst = jnp.pad(a[1].reshape(_NS, _EPS), ((0, 0), (0, _EPP - _EPS)),
                constant_values=_N)
  return jnp.stack([src, dst]).reshape(2, _NS, _NBLK, _BLK, _K)


def kernel(x, adj1, adj2, W_fc1, b_fc1, W0, b0, W1, b1, W2, b2):
  adj = jnp.stack([_pad_adj(adj1), _pad_adj(adj2)])
  zeros = jnp.zeros((_CW, _H), jnp.float32)

  h = _fc1(x, W_fc1, b_fc1.reshape(1, _H))

  degs = _prop(jnp.ones((_N, _H), jnp.float32), adj, zeros)
  inv = 1.0 / jnp.clip(degs[:, :_N, :1], 1.0, None)

  s = _prop(h, adj, zeros)
  h = _layer(s, inv, h, W0, b0.reshape(1, _H))
  s = _prop(h, adj, zeros)
  h = _layer(s, inv, h, W1, b1.reshape(1, _H))

  w2p = jnp.pad(W2, ((0, 0), (0, 64 - _C)))
  b2p = jnp.pad(b2, (0, 64 - _C)).reshape(1, 64)
  s = _prop(h, adj, zeros)
  out = _layer2(s, inv, w2p, b2p)
  return out[:, :_C]


# speedup vs baseline: 3.0591x; 1.2541x over previous
import jax
import jax.numpy as jnp
from jax import lax
from jax.experimental import pallas as pl
from jax.experimental.pallas import tpu as pltpu
from jax.experimental.pallas import tpu_sc as plsc

_N = 10000
_E = 320000
_D = 128
_H = 128
_C = 40

_NC = 2
_NS = 16
_K = 128
_BLK = 4
_NBLK = 40
_EPS = _E // _NS
_EPP = _NBLK * _BLK * _K
_NP = 10240
_RPS = _NP // _NS
_CW = 128
_NWB = _RPS // _CW

_sc_mesh = plsc.VectorSubcoreMesh(core_axis_name="c", subcore_axis_name="s")


def _make_prop():
  scratch = [
      pltpu.VMEM((_BLK, _K), jnp.int32),
      pltpu.VMEM((_BLK, _K), jnp.int32),
      pltpu.VMEM((_BLK, _K), jnp.int32),
      pltpu.VMEM((_BLK, _K), jnp.int32),
      pltpu.VMEM((_K, _H), jnp.float32),
      pltpu.VMEM((_K, _H), jnp.float32),
      pltpu.SemaphoreType.DMA,
      pltpu.SemaphoreType.DMA,
      pltpu.SemaphoreType.DMA,
      pltpu.SemaphoreType.DMA,
      pltpu.VMEM_SHARED((_NP, _H), jnp.float32),
  ]

  def body(h_hbm, adj_hbm, zeros_hbm, out_hbm,
           src0, dst0, src1, dst1, rows0, rows1,
           sem_g0, sem_g1, sem_is, sem_id, acc):
    c = lax.axis_index("c")
    s = lax.axis_index("s")
    r0 = s * _RPS
    srcs, dsts = (src0, src1), (dst0, dst1)
    rows, sem_g = (rows0, rows1), (sem_g0, sem_g1)

    pltpu.sync_copy(zeros_hbm, rows0)
    for w in range(_NWB):
      o = pl.multiple_of(r0 + w * _CW, 8)
      pltpu.sync_copy(rows0, acc.at[pl.ds(o, _CW)])
    plsc.subcore_barrier()

    pltpu.sync_copy(adj_hbm.at[c, 0, s, 0], src0)
    pltpu.sync_copy(adj_hbm.at[c, 1, s, 0], dst0)
    pltpu.async_copy(h_hbm.at[src0.at[0]], rows0, sem_g0)

    @pl.loop(0, _NBLK, step=2)
    def _pair(b):
      for pb in range(2):
        bb = b + pb
        sv, dv = srcs[pb], dsts[pb]
        nsv, ndv = srcs[pb ^ 1], dsts[pb ^ 1]
        for j in range(_BLK):
          par = j % 2
          pltpu.make_async_copy(h_hbm.at[sv.at[j]], rows[par],
                                sem_g[par]).wait()
          if j == 0:
            @pl.when(bb + 1 < _NBLK)
            def _():
              pltpu.async_copy(adj_hbm.at[c, 0, s, bb + 1], nsv, sem_is)
              pltpu.async_copy(adj_hbm.at[c, 1, s, bb + 1], ndv, sem_id)
          if j < _BLK - 1:
            pltpu.async_copy(h_hbm.at[sv.at[j + 1]], rows[par ^ 1],
                             sem_g[par ^ 1])
          else:
            @pl.when(bb + 1 < _NBLK)
            def _():
              pltpu.make_async_copy(adj_hbm.at[c, 0, s, bb + 1], nsv,
                                    sem_is).wait()
              pltpu.make_async_copy(adj_hbm.at[c, 1, s, bb + 1], ndv,
                                    sem_id).wait()
              pltpu.async_copy(h_hbm.at[nsv.at[0]], rows[par ^ 1],
                               sem_g[par ^ 1])
          pltpu.sync_copy(rows[par], acc.at[dv.at[j]], add=True)

    plsc.subcore_barrier()

    for w in range(_NWB):
      o = pl.multiple_of(r0 + w * _CW, 8)
      pltpu.sync_copy(acc.at[pl.ds(o, _CW)], rows0)
      pltpu.sync_copy(rows0, out_hbm.at[c, pl.ds(o, _CW)])

  return pl.kernel(body,
                   out_type=jax.ShapeDtypeStruct((_NC, _NP, _H), jnp.float32),
                   mesh=_sc_mesh, scratch_types=scratch)


_prop = _make_prop()


def _make_deg():
  scratch = [
      pltpu.VMEM((_BLK, _K), jnp.int32),
      pltpu.VMEM((_BLK, _K), jnp.int32),
      pltpu.VMEM((_K, _H), jnp.float32),
      pltpu.SemaphoreType.DMA,
      pltpu.SemaphoreType.DMA,
      pltpu.VMEM_SHARED((_NP, _H), jnp.float32),
  ]

  def body(ones_hbm, adj_hbm, zeros_hbm, out_hbm,
           dst0, dst1, ones_v, sem_i0, sem_i1, acc):
    c = lax.axis_index("c")
    s = lax.axis_index("s")
    r0 = s * _RPS
    dsts, sem_i = (dst0, dst1), (sem_i0, sem_i1)

    pltpu.sync_copy(zeros_hbm, ones_v)
    for w in range(_NWB):
      o = pl.multiple_of(r0 + w * _CW, 8)
      pltpu.sync_copy(ones_v, acc.at[pl.ds(o, _CW)])
    pltpu.sync_copy(ones_hbm, ones_v)
    plsc.subcore_barrier()

    pltpu.sync_copy(adj_hbm.at[c, 1, s, 0], dst0)

    @pl.loop(0, _NBLK, step=2)
    def _pair(b):
      for pb in range(2):
        bb = b + pb
        dv, ndv = dsts[pb], dsts[pb ^ 1]

        @pl.when(bb + 1 < _NBLK)
        def _():
          pltpu.async_copy(adj_hbm.at[c, 1, s, bb + 1], ndv, sem_i[pb ^ 1])

        for j in range(_BLK):
          pltpu.sync_copy(ones_v, acc.at[dv.at[j]], add=True)

        @pl.when(bb + 1 < _NBLK)
        def _():
          pltpu.make_async_copy(adj_hbm.at[c, 1, s, bb + 1], ndv,
                                sem_i[pb ^ 1]).wait()

    plsc.subcore_barrier()

    for w in range(_NWB):
      o = pl.multiple_of(r0 + w * _CW, 8)
      pltpu.sync_copy(acc.at[pl.ds(o, _CW)], ones_v)
      pltpu.sync_copy(ones_v, out_hbm.at[c, pl.ds(o, _CW)])

  return pl.kernel(body,
                   out_type=jax.ShapeDtypeStruct((_NC, _NP, _H), jnp.float32),
                   mesh=_sc_mesh, scratch_types=scratch)


_deg = _make_deg()


_BM = 2000
_GRID = _N // _BM


def _fc1_body(x_ref, w_ref, b_ref, o_ref):
  o_ref[...] = jnp.dot(x_ref[...], w_ref[...],
                       preferred_element_type=jnp.float32,
                       precision=lax.Precision.HIGHEST) + b_ref[...]


def _fc1(x, w, b):
  return pl.pallas_call(
      _fc1_body,
      grid=(_GRID,),
      in_specs=[
          pl.BlockSpec((_BM, _D), lambda i: (i, 0)),
          pl.BlockSpec((_D, _H), lambda i: (0, 0)),
          pl.BlockSpec((1, _H), lambda i: (0, 0)),
      ],
      out_specs=pl.BlockSpec((_BM, _H), lambda i: (i, 0)),
      out_shape=jax.ShapeDtypeStruct((_N, _H), jnp.float32),
  )(x, w, b)


def _layer_body(s_ref, inv_ref, h_ref, w_ref, b_ref, o_ref):
  agg = 0.5 * (s_ref[0] * inv_ref[0] + s_ref[1] * inv_ref[1])
  out = jnp.dot(agg, w_ref[...], preferred_element_type=jnp.float32,
                precision=lax.Precision.HIGHEST) + b_ref[...]
  o_ref[...] = jnp.maximum(out + h_ref[...], 0.0)


def _layer(s, inv, h, w, b):
  return pl.pallas_call(
      _layer_body,
      grid=(_GRID,),
      in_specs=[
          pl.BlockSpec((2, _BM, _H), lambda i: (0, i, 0)),
          pl.BlockSpec((2, _BM, 1), lambda i: (0, i, 0)),
          pl.BlockSpec((_BM, _H), lambda i: (i, 0)),
          pl.BlockSpec((_H, _H), lambda i: (0, 0)),
          pl.BlockSpec((1, _H), lambda i: (0, 0)),
      ],
      out_specs=pl.BlockSpec((_BM, _H), lambda i: (i, 0)),
      out_shape=jax.ShapeDtypeStruct((_N, _H), jnp.float32),
  )(s, inv, h, w, b)


def _layer2_body(s_ref, inv_ref, w_ref, b_ref, o_ref):
  agg = 0.5 * (s_ref[0] * inv_ref[0] + s_ref[1] * inv_ref[1])
  out = jnp.dot(agg, w_ref[...], preferred_element_type=jnp.float32,
                precision=lax.Precision.HIGHEST) + b_ref[...]
  o_ref[...] = jnp.maximum(out, 0.0)


def _layer2(s, inv, w, b):
  return pl.pallas_call(
      _layer2_body,
      grid=(_GRID,),
      in_specs=[
          pl.BlockSpec((2, _BM, _H), lambda i: (0, i, 0)),
          pl.BlockSpec((2, _BM, 1), lambda i: (0, i, 0)),
          pl.BlockSpec((_H, 64), lambda i: (0, 0)),
          pl.BlockSpec((1, 64), lambda i: (0, 0)),
      ],
      out_specs=pl.BlockSpec((_BM, 64), lambda i: (i, 0)),
      out_shape=jax.ShapeDtypeStruct((_N, 64), jnp.float32),
  )(s, inv, w, b)


def _pad_adj(a):
  src = jnp.pad(a[0].reshape(_NS, _EPS), ((0, 0), (0, _EPP - _EPS)))
  dst = jnp.pad(a[1].reshape(_NS, _EPS), ((0, 0), (0, _EPP - _EPS)),
                constant_values=_N)
  return jnp.stack([src, dst]).reshape(2, _NS, _NBLK, _BLK, _K)


def kernel(x, adj1, adj2, W_fc1, b_fc1, W0, b0, W1, b1, W2, b2):
  adj = jnp.stack([_pad_adj(adj1), _pad_adj(adj2)])
  zeros = jnp.zeros((_CW, _H), jnp.float32)

  h = _fc1(x, W_fc1, b_fc1.reshape(1, _H))

  degs = _deg(jnp.ones((_K, _H), jnp.float32), adj, zeros)
  inv = 1.0 / jnp.clip(degs[:, :_N, :1], 1.0, None)

  s = _prop(h, adj, zeros)
  h = _layer(s, inv, h, W0, b0.reshape(1, _H))
  s = _prop(h, adj, zeros)
  h = _layer(s, inv, h, W1, b1.reshape(1, _H))

  w2p = jnp.pad(W2, ((0, 0), (0, 64 - _C)))
  b2p = jnp.pad(b2, (0, 64 - _C)).reshape(1, 64)
  s = _prop(h, adj, zeros)
  out = _layer2(s, inv, w2p, b2p)
  return out[:, :_C]


# speedup vs baseline: 3.2135x; 1.0505x over previous
import jax
import jax.numpy as jnp
from jax import lax
from jax.experimental import pallas as pl
from jax.experimental.pallas import tpu as pltpu
from jax.experimental.pallas import tpu_sc as plsc

_N = 10000
_E = 320000
_D = 128
_H = 128
_C = 40

_NC = 2
_NS = 16
_K = 128
_BLK = 4
_NBLK = 40
_EPS = _E // _NS
_EPP = _NBLK * _BLK * _K
_NP = 10240
_RPS = _NP // _NS
_CW = 128
_NWB = _RPS // _CW

_sc_mesh = plsc.VectorSubcoreMesh(core_axis_name="c", subcore_axis_name="s")


def _make_prop():
  scratch = [
      pltpu.VMEM((_BLK, _K), jnp.int32),
      pltpu.VMEM((_BLK, _K), jnp.int32),
      pltpu.VMEM((_BLK, _K), jnp.int32),
      pltpu.VMEM((_BLK, _K), jnp.int32),
      pltpu.VMEM((_K, _H), jnp.float32),
      pltpu.VMEM((_K, _H), jnp.float32),
      pltpu.SemaphoreType.DMA,
      pltpu.SemaphoreType.DMA,
      pltpu.SemaphoreType.DMA,
      pltpu.SemaphoreType.DMA,
      pltpu.VMEM_SHARED((_NP, _H), jnp.float32),
  ]

  def body(h_hbm, adj_hbm, zeros_hbm, out_hbm,
           src0, dst0, src1, dst1, rows0, rows1,
           sem_g0, sem_g1, sem_is, sem_id, acc):
    c = lax.axis_index("c")
    s = lax.axis_index("s")
    r0 = s * _RPS
    srcs, dsts = (src0, src1), (dst0, dst1)
    rows, sem_g = (rows0, rows1), (sem_g0, sem_g1)

    pltpu.sync_copy(zeros_hbm, rows0)
    for w in range(_NWB):
      o = pl.multiple_of(r0 + w * _CW, 8)
      pltpu.sync_copy(rows0, acc.at[pl.ds(o, _CW)])
    plsc.subcore_barrier()

    pltpu.sync_copy(adj_hbm.at[c, 0, s, 0], src0)
    pltpu.sync_copy(adj_hbm.at[c, 1, s, 0], dst0)
    pltpu.async_copy(h_hbm.at[src0.at[0]], rows0, sem_g0)

    @pl.loop(0, _NBLK, step=2)
    def _pair(b):
      for pb in range(2):
        bb = b + pb
        sv, dv = srcs[pb], dsts[pb]
        nsv, ndv = srcs[pb ^ 1], dsts[pb ^ 1]
        for j in range(_BLK):
          par = j % 2
          if j == 0:
            @pl.when(bb + 1 < _NBLK)
            def _():
              pltpu.async_copy(adj_hbm.at[c, 0, s, bb + 1], nsv, sem_is)
              pltpu.async_copy(adj_hbm.at[c, 1, s, bb + 1], ndv, sem_id)
          if j < _BLK - 1:
            pltpu.async_copy(h_hbm.at[sv.at[j + 1]], rows[par ^ 1],
                             sem_g[par ^ 1])
          else:
            @pl.when(bb + 1 < _NBLK)
            def _():
              pltpu.make_async_copy(adj_hbm.at[c, 0, s, bb + 1], nsv,
                                    sem_is).wait()
              pltpu.make_async_copy(adj_hbm.at[c, 1, s, bb + 1], ndv,
                                    sem_id).wait()
              pltpu.async_copy(h_hbm.at[nsv.at[0]], rows[par ^ 1],
                               sem_g[par ^ 1])
          pltpu.make_async_copy(h_hbm.at[sv.at[j]], rows[par],
                                sem_g[par]).wait()
          pltpu.sync_copy(rows[par], acc.at[dv.at[j]], add=True)

    plsc.subcore_barrier()

    for w in range(_NWB):
      o = pl.multiple_of(r0 + w * _CW, 8)
      pltpu.sync_copy(acc.at[pl.ds(o, _CW)], rows0)
      pltpu.sync_copy(rows0, out_hbm.at[c, pl.ds(o, _CW)])

  return pl.kernel(body,
                   out_type=jax.ShapeDtypeStruct((_NC, _NP, _H), jnp.float32),
                   mesh=_sc_mesh, scratch_types=scratch)


_prop = _make_prop()


def _make_deg():
  scratch = [
      pltpu.VMEM((_BLK, _K), jnp.int32),
      pltpu.VMEM((_BLK, _K), jnp.int32),
      pltpu.VMEM((_K, _H), jnp.float32),
      pltpu.SemaphoreType.DMA,
      pltpu.SemaphoreType.DMA,
      pltpu.VMEM_SHARED((_NP, _H), jnp.float32),
  ]

  def body(ones_hbm, adj_hbm, zeros_hbm, out_hbm,
           dst0, dst1, ones_v, sem_i0, sem_i1, acc):
    c = lax.axis_index("c")
    s = lax.axis_index("s")
    r0 = s * _RPS
    dsts, sem_i = (dst0, dst1), (sem_i0, sem_i1)

    pltpu.sync_copy(zeros_hbm, ones_v)
    for w in range(_NWB):
      o = pl.multiple_of(r0 + w * _CW, 8)
      pltpu.sync_copy(ones_v, acc.at[pl.ds(o, _CW)])
    pltpu.sync_copy(ones_hbm, ones_v)
    plsc.subcore_barrier()

    pltpu.sync_copy(adj_hbm.at[c, 1, s, 0], dst0)

    @pl.loop(0, _NBLK, step=2)
    def _pair(b):
      for pb in range(2):
        bb = b + pb
        dv, ndv = dsts[pb], dsts[pb ^ 1]

        @pl.when(bb + 1 < _NBLK)
        def _():
          pltpu.async_copy(adj_hbm.at[c, 1, s, bb + 1], ndv, sem_i[pb ^ 1])

        for j in range(_BLK):
          pltpu.sync_copy(ones_v, acc.at[dv.at[j]], add=True)

        @pl.when(bb + 1 < _NBLK)
        def _():
          pltpu.make_async_copy(adj_hbm.at[c, 1, s, bb + 1], ndv,
                                sem_i[pb ^ 1]).wait()

    plsc.subcore_barrier()

    for w in range(_NWB):
      o = pl.multiple_of(r0 + w * _CW, 8)
      pltpu.sync_copy(acc.at[pl.ds(o, _CW)], ones_v)
      pltpu.sync_copy(ones_v, out_hbm.at[c, pl.ds(o, _CW)])

  return pl.kernel(body,
                   out_type=jax.ShapeDtypeStruct((_NC, _NP, _H), jnp.float32),
                   mesh=_sc_mesh, scratch_types=scratch)


_deg = _make_deg()


_BM = 2000
_GRID = _N // _BM


def _fc1_body(x_ref, w_ref, b_ref, o_ref):
  o_ref[...] = jnp.dot(x_ref[...], w_ref[...],
                       preferred_element_type=jnp.float32,
                       precision=lax.Precision.HIGHEST) + b_ref[...]


def _fc1(x, w, b):
  return pl.pallas_call(
      _fc1_body,
      grid=(_GRID,),
      in_specs=[
          pl.BlockSpec((_BM, _D), lambda i: (i, 0)),
          pl.BlockSpec((_D, _H), lambda i: (0, 0)),
          pl.BlockSpec((1, _H), lambda i: (0, 0)),
      ],
      out_specs=pl.BlockSpec((_BM, _H), lambda i: (i, 0)),
      out_shape=jax.ShapeDtypeStruct((_N, _H), jnp.float32),
  )(x, w, b)


def _layer_body(s_ref, inv_ref, h_ref, w_ref, b_ref, o_ref):
  agg = 0.5 * (s_ref[0] * inv_ref[0] + s_ref[1] * inv_ref[1])
  out = jnp.dot(agg, w_ref[...], preferred_element_type=jnp.float32,
                precision=lax.Precision.HIGHEST) + b_ref[...]
  o_ref[...] = jnp.maximum(out + h_ref[...], 0.0)


def _layer(s, inv, h, w, b):
  return pl.pallas_call(
      _layer_body,
      grid=(_GRID,),
      in_specs=[
          pl.BlockSpec((2, _BM, _H), lambda i: (0, i, 0)),
          pl.BlockSpec((2, _BM, 1), lambda i: (0, i, 0)),
          pl.BlockSpec((_BM, _H), lambda i: (i, 0)),
          pl.BlockSpec((_H, _H), lambda i: (0, 0)),
          pl.BlockSpec((1, _H), lambda i: (0, 0)),
      ],
      out_specs=pl.BlockSpec((_BM, _H), lambda i: (i, 0)),
      out_shape=jax.ShapeDtypeStruct((_N, _H), jnp.float32),
  )(s, inv, h, w, b)


def _layer2_body(s_ref, inv_ref, w_ref, b_ref, o_ref):
  agg = 0.5 * (s_ref[0] * inv_ref[0] + s_ref[1] * inv_ref[1])
  out = jnp.dot(agg, w_ref[...], preferred_element_type=jnp.float32,
                precision=lax.Precision.HIGHEST) + b_ref[...]
  o_ref[...] = jnp.maximum(out, 0.0)


def _layer2(s, inv, w, b):
  return pl.pallas_call(
      _layer2_body,
      grid=(_GRID,),
      in_specs=[
          pl.BlockSpec((2, _BM, _H), lambda i: (0, i, 0)),
          pl.BlockSpec((2, _BM, 1), lambda i: (0, i, 0)),
          pl.BlockSpec((_H, 64), lambda i: (0, 0)),
          pl.BlockSpec((1, 64), lambda i: (0, 0)),
      ],
      out_specs=pl.BlockSpec((_BM, 64), lambda i: (i, 0)),
      out_shape=jax.ShapeDtypeStruct((_N, 64), jnp.float32),
  )(s, inv, w, b)


def _pad_adj(a):
  src = jnp.pad(a[0].reshape(_NS, _EPS), ((0, 0), (0, _EPP - _EPS)))
  dst = jnp.pad(a[1].reshape(_NS, _EPS), ((0, 0), (0, _EPP - _EPS)),
                constant_values=_N)
  return jnp.stack([src, dst]).reshape(2, _NS, _NBLK, _BLK, _K)


def kernel(x, adj1, adj2, W_fc1, b_fc1, W0, b0, W1, b1, W2, b2):
  adj = jnp.stack([_pad_adj(adj1), _pad_adj(adj2)])
  zeros = jnp.zeros((_CW, _H), jnp.float32)

  h = _fc1(x, W_fc1, b_fc1.reshape(1, _H))

  degs = _deg(jnp.ones((_K, _H), jnp.float32), adj, zeros)
  inv = 1.0 / jnp.clip(degs[:, :_N, :1], 1.0, None)

  s = _prop(h, adj, zeros)
  h = _layer(s, inv, h, W0, b0.reshape(1, _H))
  s = _prop(h, adj, zeros)
  h = _layer(s, inv, h, W1, b1.reshape(1, _H))

  w2p = jnp.pad(W2, ((0, 0), (0, 64 - _C)))
  b2p = jnp.pad(b2, (0, 64 - _C)).reshape(1, 64)
  s = _prop(h, adj, zeros)
  out = _layer2(s, inv, w2p, b2p)
  return out[:, :_C]


# speedup vs baseline: 3.2162x; 1.0008x over previous
import jax
import jax.numpy as jnp
from jax import lax
from jax.experimental import pallas as pl
from jax.experimental.pallas import tpu as pltpu
from jax.experimental.pallas import tpu_sc as plsc

_N = 10000
_E = 320000
_D = 128
_H = 128
_C = 40

_NC = 2
_NS = 16
_K = 128
_BLK = 4
_NBLK = 40
_EPS = _E // _NS
_EPP = _NBLK * _BLK * _K
_NP = 10240
_RPS = _NP // _NS
_CW = 128
_NWB = _RPS // _CW

_sc_mesh = plsc.VectorSubcoreMesh(core_axis_name="c", subcore_axis_name="s")


def _make_prop():
  scratch = [
      pltpu.VMEM((_BLK, _K), jnp.int32),
      pltpu.VMEM((_BLK, _K), jnp.int32),
      pltpu.VMEM((_BLK, _K), jnp.int32),
      pltpu.VMEM((_BLK, _K), jnp.int32),
      pltpu.VMEM((_K, _H), jnp.float32),
      pltpu.VMEM((_K, _H), jnp.float32),
      pltpu.SemaphoreType.DMA,
      pltpu.SemaphoreType.DMA,
      pltpu.SemaphoreType.DMA,
      pltpu.SemaphoreType.DMA,
      pltpu.VMEM_SHARED((_NP, _H), jnp.float32),
  ]

  def body(h_hbm, adj_hbm, zeros_hbm, out_hbm,
           src0, dst0, src1, dst1, rows0, rows1,
           sem_g0, sem_g1, sem_is, sem_id, acc):
    c = lax.axis_index("c")
    s = lax.axis_index("s")
    r0 = s * _RPS
    srcs, dsts = (src0, src1), (dst0, dst1)
    rows, sem_g = (rows0, rows1), (sem_g0, sem_g1)

    pltpu.sync_copy(zeros_hbm, rows0)
    for w in range(_NWB):
      o = pl.multiple_of(r0 + w * _CW, 8)
      pltpu.sync_copy(rows0, acc.at[pl.ds(o, _CW)])
    plsc.subcore_barrier()

    pltpu.sync_copy(adj_hbm.at[c, 0, s, 0], src0)
    pltpu.sync_copy(adj_hbm.at[c, 1, s, 0], dst0)
    pltpu.async_copy(h_hbm.at[src0.at[0, pl.ds(0, 64)]],
                     rows0.at[pl.ds(0, 64)], sem_g0)
    pltpu.async_copy(h_hbm.at[src0.at[0, pl.ds(64, 64)]],
                     rows0.at[pl.ds(64, 64)], sem_g0)

    @pl.loop(0, _NBLK, step=2)
    def _pair(b):
      for pb in range(2):
        bb = b + pb
        sv, dv = srcs[pb], dsts[pb]
        nsv, ndv = srcs[pb ^ 1], dsts[pb ^ 1]
        for j in range(_BLK):
          par = j % 2
          if j == 0:
            @pl.when(bb + 1 < _NBLK)
            def _():
              pltpu.async_copy(adj_hbm.at[c, 0, s, bb + 1], nsv, sem_is)
              pltpu.async_copy(adj_hbm.at[c, 1, s, bb + 1], ndv, sem_id)
          if j < _BLK - 1:
            pltpu.async_copy(h_hbm.at[sv.at[j + 1, pl.ds(0, 64)]],
                             rows[par ^ 1].at[pl.ds(0, 64)], sem_g[par ^ 1])
            pltpu.async_copy(h_hbm.at[sv.at[j + 1, pl.ds(64, 64)]],
                             rows[par ^ 1].at[pl.ds(64, 64)], sem_g[par ^ 1])
          else:
            @pl.when(bb + 1 < _NBLK)
            def _():
              pltpu.make_async_copy(adj_hbm.at[c, 0, s, bb + 1], nsv,
                                    sem_is).wait()
              pltpu.make_async_copy(adj_hbm.at[c, 1, s, bb + 1], ndv,
                                    sem_id).wait()
              pltpu.async_copy(h_hbm.at[nsv.at[0, pl.ds(0, 64)]],
                               rows[par ^ 1].at[pl.ds(0, 64)], sem_g[par ^ 1])
              pltpu.async_copy(h_hbm.at[nsv.at[0, pl.ds(64, 64)]],
                               rows[par ^ 1].at[pl.ds(64, 64)], sem_g[par ^ 1])
          pltpu.make_async_copy(h_hbm.at[sv.at[j, pl.ds(0, 64)]],
                                rows[par].at[pl.ds(0, 64)], sem_g[par]).wait()
          pltpu.make_async_copy(h_hbm.at[sv.at[j, pl.ds(64, 64)]],
                                rows[par].at[pl.ds(64, 64)], sem_g[par]).wait()
          pltpu.sync_copy(rows[par], acc.at[dv.at[j]], add=True)

    plsc.subcore_barrier()

    for w in range(_NWB):
      o = pl.multiple_of(r0 + w * _CW, 8)
      pltpu.sync_copy(acc.at[pl.ds(o, _CW)], rows0)
      pltpu.sync_copy(rows0, out_hbm.at[c, pl.ds(o, _CW)])

  return pl.kernel(body,
                   out_type=jax.ShapeDtypeStruct((_NC, _NP, _H), jnp.float32),
                   mesh=_sc_mesh, scratch_types=scratch)


_prop = _make_prop()


def _make_deg():
  scratch = [
      pltpu.VMEM((_BLK, _K), jnp.int32),
      pltpu.VMEM((_BLK, _K), jnp.int32),
      pltpu.VMEM((_K, _H), jnp.float32),
      pltpu.SemaphoreType.DMA,
      pltpu.SemaphoreType.DMA,
      pltpu.VMEM_SHARED((_NP, _H), jnp.float32),
  ]

  def body(ones_hbm, adj_hbm, zeros_hbm, out_hbm,
           dst0, dst1, ones_v, sem_i0, sem_i1, acc):
    c = lax.axis_index("c")
    s = lax.axis_index("s")
    r0 = s * _RPS
    dsts, sem_i = (dst0, dst1), (sem_i0, sem_i1)

    pltpu.sync_copy(zeros_hbm, ones_v)
    for w in range(_NWB):
      o = pl.multiple_of(r0 + w * _CW, 8)
      pltpu.sync_copy(ones_v, acc.at[pl.ds(o, _CW)])
    pltpu.sync_copy(ones_hbm, ones_v)
    plsc.subcore_barrier()

    pltpu.sync_copy(adj_hbm.at[c, 1, s, 0], dst0)

    @pl.loop(0, _NBLK, step=2)
    def _pair(b):
      for pb in range(2):
        bb = b + pb
        dv, ndv = dsts[pb], dsts[pb ^ 1]

        @pl.when(bb + 1 < _NBLK)
        def _():
          pltpu.async_copy(adj_hbm.at[c, 1, s, bb + 1], ndv, sem_i[pb ^ 1])

        for j in range(_BLK):
          pltpu.sync_copy(ones_v, acc.at[dv.at[j]], add=True)

        @pl.when(bb + 1 < _NBLK)
        def _():
          pltpu.make_async_copy(adj_hbm.at[c, 1, s, bb + 1], ndv,
                                sem_i[pb ^ 1]).wait()

    plsc.subcore_barrier()

    for w in range(_NWB):
      o = pl.multiple_of(r0 + w * _CW, 8)
      pltpu.sync_copy(acc.at[pl.ds(o, _CW)], ones_v)
      pltpu.sync_copy(ones_v, out_hbm.at[c, pl.ds(o, _CW)])

  return pl.kernel(body,
                   out_type=jax.ShapeDtypeStruct((_NC, _NP, _H), jnp.float32),
                   mesh=_sc_mesh, scratch_types=scratch)


_deg = _make_deg()


_BM = 2000
_GRID = _N // _BM


def _fc1_body(x_ref, w_ref, b_ref, o_ref):
  o_ref[...] = jnp.dot(x_ref[...], w_ref[...],
                       preferred_element_type=jnp.float32,
                       precision=lax.Precision.HIGHEST) + b_ref[...]


def _fc1(x, w, b):
  return pl.pallas_call(
      _fc1_body,
      grid=(_GRID,),
      in_specs=[
          pl.BlockSpec((_BM, _D), lambda i: (i, 0)),
          pl.BlockSpec((_D, _H), lambda i: (0, 0)),
          pl.BlockSpec((1, _H), lambda i: (0, 0)),
      ],
      out_specs=pl.BlockSpec((_BM, _H), lambda i: (i, 0)),
      out_shape=jax.ShapeDtypeStruct((_N, _H), jnp.float32),
  )(x, w, b)


def _layer_body(s_ref, inv_ref, h_ref, w_ref, b_ref, o_ref):
  agg = 0.5 * (s_ref[0] * inv_ref[0] + s_ref[1] * inv_ref[1])
  out = jnp.dot(agg, w_ref[...], preferred_element_type=jnp.float32,
                precision=lax.Precision.HIGHEST) + b_ref[...]
  o_ref[...] = jnp.maximum(out + h_ref[...], 0.0)


def _layer(s, inv, h, w, b):
  return pl.pallas_call(
      _layer_body,
      grid=(_GRID,),
      in_specs=[
          pl.BlockSpec((2, _BM, _H), lambda i: (0, i, 0)),
          pl.BlockSpec((2, _BM, 1), lambda i: (0, i, 0)),
          pl.BlockSpec((_BM, _H), lambda i: (i, 0)),
          pl.BlockSpec((_H, _H), lambda i: (0, 0)),
          pl.BlockSpec((1, _H), lambda i: (0, 0)),
      ],
      out_specs=pl.BlockSpec((_BM, _H), lambda i: (i, 0)),
      out_shape=jax.ShapeDtypeStruct((_N, _H), jnp.float32),
  )(s, inv, h, w, b)


def _layer2_body(s_ref, inv_ref, w_ref, b_ref, o_ref):
  agg = 0.5 * (s_ref[0] * inv_ref[0] + s_ref[1] * inv_ref[1])
  out = jnp.dot(agg, w_ref[...], preferred_element_type=jnp.float32,
                precision=lax.Precision.HIGHEST) + b_ref[...]
  o_ref[...] = jnp.maximum(out, 0.0)


def _layer2(s, inv, w, b):
  return pl.pallas_call(
      _layer2_body,
      grid=(_GRID,),
      in_specs=[
          pl.BlockSpec((2, _BM, _H), lambda i: (0, i, 0)),
          pl.BlockSpec((2, _BM, 1), lambda i: (0, i, 0)),
          pl.BlockSpec((_H, 64), lambda i: (0, 0)),
          pl.BlockSpec((1, 64), lambda i: (0, 0)),
      ],
      out_specs=pl.BlockSpec((_BM, 64), lambda i: (i, 0)),
      out_shape=jax.ShapeDtypeStruct((_N, 64), jnp.float32),
  )(s, inv, w, b)


def _pad_adj(a):
  src = jnp.pad(a[0].reshape(_NS, _EPS), ((0, 0), (0, _EPP - _EPS)))
  dst = jnp.pad(a[1].reshape(_NS, _EPS), ((0, 0), (0, _EPP - _EPS)),
                constant_values=_N)
  return jnp.stack([src, dst]).reshape(2, _NS, _NBLK, _BLK, _K)


def kernel(x, adj1, adj2, W_fc1, b_fc1, W0, b0, W1, b1, W2, b2):
  adj = jnp.stack([_pad_adj(adj1), _pad_adj(adj2)])
  zeros = jnp.zeros((_CW, _H), jnp.float32)

  h = _fc1(x, W_fc1, b_fc1.reshape(1, _H))

  degs = _deg(jnp.ones((_K, _H), jnp.float32), adj, zeros)
  inv = 1.0 / jnp.clip(degs[:, :_N, :1], 1.0, None)

  s = _prop(h, adj, zeros)
  h = _layer(s, inv, h, W0, b0.reshape(1, _H))
  s = _prop(h, adj, zeros)
  h = _layer(s, inv, h, W1, b1.reshape(1, _H))

  w2p = jnp.pad(W2, ((0, 0), (0, 64 - _C)))
  b2p = jnp.pad(b2, (0, 64 - _C)).reshape(1, 64)
  s = _prop(h, adj, zeros)
  out = _layer2(s, inv, w2p, b2p)
  return out[:, :_C]


# speedup vs baseline: 3.2175x; 1.0004x over previous
import jax
import jax.numpy as jnp
from jax import lax
from jax.experimental import pallas as pl
from jax.experimental.pallas import tpu as pltpu
from jax.experimental.pallas import tpu_sc as plsc

_N = 10000
_E = 320000
_D = 128
_H = 128
_C = 40

_NC = 2
_NS = 16
_K = 128
_BLK = 8
_NBLK = 20
_EPS = _E // _NS
_EPP = _NBLK * _BLK * _K
_NP = 10240
_RPS = _NP // _NS
_CW = 128
_NWB = _RPS // _CW

_sc_mesh = plsc.VectorSubcoreMesh(core_axis_name="c", subcore_axis_name="s")


def _make_prop():
  scratch = [
      pltpu.VMEM((_BLK, _K), jnp.int32),
      pltpu.VMEM((_BLK, _K), jnp.int32),
      pltpu.VMEM((_BLK, _K), jnp.int32),
      pltpu.VMEM((_BLK, _K), jnp.int32),
      pltpu.VMEM((_K, _H), jnp.float32),
      pltpu.VMEM((_K, _H), jnp.float32),
      pltpu.SemaphoreType.DMA,
      pltpu.SemaphoreType.DMA,
      pltpu.SemaphoreType.DMA,
      pltpu.SemaphoreType.DMA,
      pltpu.VMEM_SHARED((_NP, _H), jnp.float32),
  ]

  def body(h_hbm, adj_hbm, zeros_hbm, out_hbm,
           src0, dst0, src1, dst1, rows0, rows1,
           sem_g0, sem_g1, sem_is, sem_id, acc):
    c = lax.axis_index("c")
    s = lax.axis_index("s")
    r0 = s * _RPS
    srcs, dsts = (src0, src1), (dst0, dst1)
    rows, sem_g = (rows0, rows1), (sem_g0, sem_g1)

    pltpu.sync_copy(zeros_hbm, rows0)
    for w in range(_NWB):
      o = pl.multiple_of(r0 + w * _CW, 8)
      pltpu.sync_copy(rows0, acc.at[pl.ds(o, _CW)])
    plsc.subcore_barrier()

    pltpu.sync_copy(adj_hbm.at[c, 0, s, 0], src0)
    pltpu.sync_copy(adj_hbm.at[c, 1, s, 0], dst0)
    pltpu.async_copy(h_hbm.at[src0.at[0]], rows0, sem_g0)

    @pl.loop(0, _NBLK, step=2)
    def _pair(b):
      for pb in range(2):
        bb = b + pb
        sv, dv = srcs[pb], dsts[pb]
        nsv, ndv = srcs[pb ^ 1], dsts[pb ^ 1]
        for j in range(_BLK):
          par = j % 2
          if j == 0:
            @pl.when(bb + 1 < _NBLK)
            def _():
              pltpu.async_copy(adj_hbm.at[c, 0, s, bb + 1], nsv, sem_is)
              pltpu.async_copy(adj_hbm.at[c, 1, s, bb + 1], ndv, sem_id)
          if j < _BLK - 1:
            pltpu.async_copy(h_hbm.at[sv.at[j + 1]], rows[par ^ 1],
                             sem_g[par ^ 1])
          else:
            @pl.when(bb + 1 < _NBLK)
            def _():
              pltpu.make_async_copy(adj_hbm.at[c, 0, s, bb + 1], nsv,
                                    sem_is).wait()
              pltpu.make_async_copy(adj_hbm.at[c, 1, s, bb + 1], ndv,
                                    sem_id).wait()
              pltpu.async_copy(h_hbm.at[nsv.at[0]], rows[par ^ 1],
                               sem_g[par ^ 1])
          pltpu.make_async_copy(h_hbm.at[sv.at[j]], rows[par],
                                sem_g[par]).wait()
          pltpu.sync_copy(rows[par], acc.at[dv.at[j]], add=True)

    plsc.subcore_barrier()

    for w in range(_NWB):
      o = pl.multiple_of(r0 + w * _CW, 8)
      pltpu.sync_copy(acc.at[pl.ds(o, _CW)], rows0)
      pltpu.sync_copy(rows0, out_hbm.at[c, pl.ds(o, _CW)])

  return pl.kernel(body,
                   out_type=jax.ShapeDtypeStruct((_NC, _NP, _H), jnp.float32),
                   mesh=_sc_mesh, scratch_types=scratch)


_prop = _make_prop()


def _make_deg():
  scratch = [
      pltpu.VMEM((_BLK, _K), jnp.int32),
      pltpu.VMEM((_BLK, _K), jnp.int32),
      pltpu.VMEM((_K, _H), jnp.float32),
      pltpu.SemaphoreType.DMA,
      pltpu.SemaphoreType.DMA,
      pltpu.VMEM_SHARED((_NP, _H), jnp.float32),
  ]

  def body(ones_hbm, adj_hbm, zeros_hbm, out_hbm,
           dst0, dst1, ones_v, sem_i0, sem_i1, acc):
    c = lax.axis_index("c")
    s = lax.axis_index("s")
    r0 = s * _RPS
    dsts, sem_i = (dst0, dst1), (sem_i0, sem_i1)

    pltpu.sync_copy(zeros_hbm, ones_v)
    for w in range(_NWB):
      o = pl.multiple_of(r0 + w * _CW, 8)
      pltpu.sync_copy(ones_v, acc.at[pl.ds(o, _CW)])
    pltpu.sync_copy(ones_hbm, ones_v)
    plsc.subcore_barrier()

    pltpu.sync_copy(adj_hbm.at[c, 1, s, 0], dst0)

    @pl.loop(0, _NBLK, step=2)
    def _pair(b):
      for pb in range(2):
        bb = b + pb
        dv, ndv = dsts[pb], dsts[pb ^ 1]

        @pl.when(bb + 1 < _NBLK)
        def _():
          pltpu.async_copy(adj_hbm.at[c, 1, s, bb + 1], ndv, sem_i[pb ^ 1])

        for j in range(_BLK):
          pltpu.sync_copy(ones_v, acc.at[dv.at[j]], add=True)

        @pl.when(bb + 1 < _NBLK)
        def _():
          pltpu.make_async_copy(adj_hbm.at[c, 1, s, bb + 1], ndv,
                                sem_i[pb ^ 1]).wait()

    plsc.subcore_barrier()

    for w in range(_NWB):
      o = pl.multiple_of(r0 + w * _CW, 8)
      pltpu.sync_copy(acc.at[pl.ds(o, _CW)], ones_v)
      pltpu.sync_copy(ones_v, out_hbm.at[c, pl.ds(o, _CW)])

  return pl.kernel(body,
                   out_type=jax.ShapeDtypeStruct((_NC, _NP, _H), jnp.float32),
                   mesh=_sc_mesh, scratch_types=scratch)


_deg = _make_deg()


_BM = 2000
_GRID = _N // _BM


def _fc1_body(x_ref, w_ref, b_ref, o_ref):
  o_ref[...] = jnp.dot(x_ref[...], w_ref[...],
                       preferred_element_type=jnp.float32,
                       precision=lax.Precision.HIGHEST) + b_ref[...]


def _fc1(x, w, b):
  return pl.pallas_call(
      _fc1_body,
      grid=(_GRID,),
      in_specs=[
          pl.BlockSpec((_BM, _D), lambda i: (i, 0)),
          pl.BlockSpec((_D, _H), lambda i: (0, 0)),
          pl.BlockSpec((1, _H), lambda i: (0, 0)),
      ],
      out_specs=pl.BlockSpec((_BM, _H), lambda i: (i, 0)),
      out_shape=jax.ShapeDtypeStruct((_N, _H), jnp.float32),
  )(x, w, b)


def _layer_body(s_ref, inv_ref, h_ref, w_ref, b_ref, o_ref):
  agg = 0.5 * (s_ref[0] * inv_ref[0] + s_ref[1] * inv_ref[1])
  out = jnp.dot(agg, w_ref[...], preferred_element_type=jnp.float32,
                precision=lax.Precision.HIGHEST) + b_ref[...]
  o_ref[...] = jnp.maximum(out + h_ref[...], 0.0)


def _layer(s, inv, h, w, b):
  return pl.pallas_call(
      _layer_body,
      grid=(_GRID,),
      in_specs=[
          pl.BlockSpec((2, _BM, _H), lambda i: (0, i, 0)),
          pl.BlockSpec((2, _BM, 1), lambda i: (0, i, 0)),
          pl.BlockSpec((_BM, _H), lambda i: (i, 0)),
          pl.BlockSpec((_H, _H), lambda i: (0, 0)),
          pl.BlockSpec((1, _H), lambda i: (0, 0)),
      ],
      out_specs=pl.BlockSpec((_BM, _H), lambda i: (i, 0)),
      out_shape=jax.ShapeDtypeStruct((_N, _H), jnp.float32),
  )(s, inv, h, w, b)


def _layer2_body(s_ref, inv_ref, w_ref, b_ref, o_ref):
  agg = 0.5 * (s_ref[0] * inv_ref[0] + s_ref[1] * inv_ref[1])
  out = jnp.dot(agg, w_ref[...], preferred_element_type=jnp.float32,
                precision=lax.Precision.HIGHEST) + b_ref[...]
  o_ref[...] = jnp.maximum(out, 0.0)


def _layer2(s, inv, w, b):
  return pl.pallas_call(
      _layer2_body,
      grid=(_GRID,),
      in_specs=[
          pl.BlockSpec((2, _BM, _H), lambda i: (0, i, 0)),
          pl.BlockSpec((2, _BM, 1), lambda i: (0, i, 0)),
          pl.BlockSpec((_H, 64), lambda i: (0, 0)),
          pl.BlockSpec((1, 64), lambda i: (0, 0)),
      ],
      out_specs=pl.BlockSpec((_BM, 64), lambda i: (i, 0)),
      out_shape=jax.ShapeDtypeStruct((_N, 64), jnp.float32),
  )(s, inv, w, b)


def _pad_adj(a):
  src = jnp.pad(a[0].reshape(_NS, _EPS), ((0, 0), (0, _EPP - _EPS)))
  dst = jnp.pad(a[1].reshape(_NS, _EPS), ((0, 0), (0, _EPP - _EPS)),
                constant_values=_N)
  return jnp.stack([src, dst]).reshape(2, _NS, _NBLK, _BLK, _K)


def kernel(x, adj1, adj2, W_fc1, b_fc1, W0, b0, W1, b1, W2, b2):
  adj = jnp.stack([_pad_adj(adj1), _pad_adj(adj2)])
  zeros = jnp.zeros((_CW, _H), jnp.float32)

  h = _fc1(x, W_fc1, b_fc1.reshape(1, _H))

  degs = _deg(jnp.ones((_K, _H), jnp.float32), adj, zeros)
  inv = 1.0 / jnp.clip(degs[:, :_N, :1], 1.0, None)

  s = _prop(h, adj, zeros)
  h = _layer(s, inv, h, W0, b0.reshape(1, _H))
  s = _prop(h, adj, zeros)
  h = _layer(s, inv, h, W1, b1.reshape(1, _H))

  w2p = jnp.pad(W2, ((0, 0), (0, 64 - _C)))
  b2p = jnp.pad(b2, (0, 64 - _C)).reshape(1, 64)
  s = _prop(h, adj, zeros)
  out = _layer2(s, inv, w2p, b2p)
  return out[:, :_C]


# speedup vs baseline: 3.2258x; 1.0026x over previous
import jax
import jax.numpy as jnp
from jax import lax
from jax.experimental import pallas as pl
from jax.experimental.pallas import tpu as pltpu
from jax.experimental.pallas import tpu_sc as plsc

_N = 10000
_E = 320000
_D = 128
_H = 128
_C = 40

_NC = 2
_NS = 16
_K = 128
_BLK = 8
_NBLK = 20
_EPS = _E // _NS
_EPP = _NBLK * _BLK * _K
_NP = 10240
_RPS = _NP // _NS
_CW = 128
_NWB = _RPS // _CW

_sc_mesh = plsc.VectorSubcoreMesh(core_axis_name="c", subcore_axis_name="s")


def _make_prop():
  scratch = [
      pltpu.VMEM((_BLK, _K), jnp.int32),
      pltpu.VMEM((_BLK, _K), jnp.int32),
      pltpu.VMEM((_BLK, _K), jnp.int32),
      pltpu.VMEM((_BLK, _K), jnp.int32),
      pltpu.VMEM((_K, _H), jnp.float32),
      pltpu.VMEM((_K, _H), jnp.float32),
      pltpu.SemaphoreType.DMA,
      pltpu.SemaphoreType.DMA,
      pltpu.SemaphoreType.DMA,
      pltpu.SemaphoreType.DMA,
      pltpu.VMEM_SHARED((_NP, _H), jnp.float32),
  ]

  def body(h_hbm, adj_hbm, zeros_hbm, out_hbm,
           src0, dst0, src1, dst1, rows0, rows1,
           sem_g0, sem_g1, sem_is, sem_id, acc):
    c = lax.axis_index("c")
    s = lax.axis_index("s")
    r0 = s * _RPS
    srcs, dsts = (src0, src1), (dst0, dst1)
    rows, sem_g = (rows0, rows1), (sem_g0, sem_g1)

    pltpu.sync_copy(zeros_hbm, rows0)
    for w in range(_NWB):
      o = pl.multiple_of(r0 + w * _CW, 8)
      pltpu.sync_copy(rows0, acc.at[pl.ds(o, _CW)])
    plsc.subcore_barrier()

    pltpu.sync_copy(adj_hbm.at[c, 0, s, 0], src0)
    pltpu.sync_copy(adj_hbm.at[c, 1, s, 0], dst0)
    pltpu.async_copy(h_hbm.at[src0.at[0]], rows0, sem_g0)

    @pl.loop(0, _NBLK, step=2)
    def _pair(b):
      for pb in range(2):
        bb = b + pb
        sv, dv = srcs[pb], dsts[pb]
        nsv, ndv = srcs[pb ^ 1], dsts[pb ^ 1]
        for j in range(_BLK):
          par = j % 2
          if j == 0:
            @pl.when(bb + 1 < _NBLK)
            def _():
              pltpu.async_copy(adj_hbm.at[c, 0, s, bb + 1], nsv, sem_is)
              pltpu.async_copy(adj_hbm.at[c, 1, s, bb + 1], ndv, sem_id)
          if j < _BLK - 1:
            pltpu.async_copy(h_hbm.at[sv.at[j + 1]], rows[par ^ 1],
                             sem_g[par ^ 1])
          else:
            @pl.when(bb + 1 < _NBLK)
            def _():
              pltpu.make_async_copy(adj_hbm.at[c, 0, s, bb + 1], nsv,
                                    sem_is).wait()
              pltpu.make_async_copy(adj_hbm.at[c, 1, s, bb + 1], ndv,
                                    sem_id).wait()
              pltpu.async_copy(h_hbm.at[nsv.at[0]], rows[par ^ 1],
                               sem_g[par ^ 1])
          pltpu.make_async_copy(h_hbm.at[sv.at[j]], rows[par],
                                sem_g[par]).wait()
          pltpu.sync_copy(rows[par], acc.at[dv.at[j]], add=True)

    plsc.subcore_barrier()

    for w in range(_NWB):
      o = pl.multiple_of(r0 + w * _CW, 8)
      pltpu.sync_copy(acc.at[pl.ds(o, _CW)], rows0)
      pltpu.sync_copy(rows0, out_hbm.at[c, pl.ds(o, _CW)])

  return pl.kernel(body,
                   out_type=jax.ShapeDtypeStruct((_NC, _NP, _H), jnp.float32),
                   mesh=_sc_mesh, scratch_types=scratch)


_prop = _make_prop()


def _make_deg():
  scratch = [
      pltpu.VMEM((_BLK, _K), jnp.int32),
      pltpu.VMEM((_BLK, _K), jnp.int32),
      pltpu.VMEM((_K, _H), jnp.float32),
      pltpu.SemaphoreType.DMA,
      pltpu.SemaphoreType.DMA,
      pltpu.VMEM_SHARED((_NP, _H), jnp.float32),
  ]

  def body(ones_hbm, adj_hbm, zeros_hbm, out_hbm,
           dst0, dst1, ones_v, sem_i0, sem_i1, acc):
    c = lax.axis_index("c")
    s = lax.axis_index("s")
    r0 = s * _RPS
    dsts, sem_i = (dst0, dst1), (sem_i0, sem_i1)

    pltpu.sync_copy(zeros_hbm, ones_v)
    for w in range(_NWB):
      o = pl.multiple_of(r0 + w * _CW, 8)
      pltpu.sync_copy(ones_v, acc.at[pl.ds(o, _CW)])
    pltpu.sync_copy(ones_hbm, ones_v)
    plsc.subcore_barrier()

    pltpu.sync_copy(adj_hbm.at[c, 1, s, 0], dst0)

    @pl.loop(0, _NBLK, step=2)
    def _pair(b):
      for pb in range(2):
        bb = b + pb
        dv, ndv = dsts[pb], dsts[pb ^ 1]

        @pl.when(bb + 1 < _NBLK)
        def _():
          pltpu.async_copy(adj_hbm.at[c, 1, s, bb + 1], ndv, sem_i[pb ^ 1])

        for j in range(_BLK):
          pltpu.sync_copy(ones_v, acc.at[dv.at[j]], add=True)

        @pl.when(bb + 1 < _NBLK)
        def _():
          pltpu.make_async_copy(adj_hbm.at[c, 1, s, bb + 1], ndv,
                                sem_i[pb ^ 1]).wait()

    plsc.subcore_barrier()

    for w in range(_NWB):
      o = pl.multiple_of(r0 + w * _CW, 8)
      pltpu.sync_copy(acc.at[pl.ds(o, _CW)], ones_v)
      pltpu.sync_copy(ones_v, out_hbm.at[c, pl.ds(o, _CW)])

  return pl.kernel(body,
                   out_type=jax.ShapeDtypeStruct((_NC, _NP, _H), jnp.float32),
                   mesh=_sc_mesh, scratch_types=scratch)


_deg = _make_deg()


_BM = 2000
_GRID = _N // _BM


def _fc1_body(x_ref, w_ref, b_ref, o_ref):
  o_ref[...] = jnp.dot(x_ref[...], w_ref[...],
                       preferred_element_type=jnp.float32,
                       precision=lax.Precision.DEFAULT) + b_ref[...]


def _fc1(x, w, b):
  return pl.pallas_call(
      _fc1_body,
      grid=(_GRID,),
      in_specs=[
          pl.BlockSpec((_BM, _D), lambda i: (i, 0)),
          pl.BlockSpec((_D, _H), lambda i: (0, 0)),
          pl.BlockSpec((1, _H), lambda i: (0, 0)),
      ],
      out_specs=pl.BlockSpec((_BM, _H), lambda i: (i, 0)),
      out_shape=jax.ShapeDtypeStruct((_N, _H), jnp.float32),
  )(x, w, b)


def _layer_body(s_ref, inv_ref, h_ref, w_ref, b_ref, o_ref):
  agg = 0.5 * (s_ref[0] * inv_ref[0] + s_ref[1] * inv_ref[1])
  out = jnp.dot(agg, w_ref[...], preferred_element_type=jnp.float32,
                precision=lax.Precision.DEFAULT) + b_ref[...]
  o_ref[...] = jnp.maximum(out + h_ref[...], 0.0)


def _layer(s, inv, h, w, b):
  return pl.pallas_call(
      _layer_body,
      grid=(_GRID,),
      in_specs=[
          pl.BlockSpec((2, _BM, _H), lambda i: (0, i, 0)),
          pl.BlockSpec((2, _BM, 1), lambda i: (0, i, 0)),
          pl.BlockSpec((_BM, _H), lambda i: (i, 0)),
          pl.BlockSpec((_H, _H), lambda i: (0, 0)),
          pl.BlockSpec((1, _H), lambda i: (0, 0)),
      ],
      out_specs=pl.BlockSpec((_BM, _H), lambda i: (i, 0)),
      out_shape=jax.ShapeDtypeStruct((_N, _H), jnp.float32),
  )(s, inv, h, w, b)


def _layer2_body(s_ref, inv_ref, w_ref, b_ref, o_ref):
  agg = 0.5 * (s_ref[0] * inv_ref[0] + s_ref[1] * inv_ref[1])
  out = jnp.dot(agg, w_ref[...], preferred_element_type=jnp.float32,
                precision=lax.Precision.DEFAULT) + b_ref[...]
  o_ref[...] = jnp.maximum(out, 0.0)


def _layer2(s, inv, w, b):
  return pl.pallas_call(
      _layer2_body,
      grid=(_GRID,),
      in_specs=[
          pl.BlockSpec((2, _BM, _H), lambda i: (0, i, 0)),
          pl.BlockSpec((2, _BM, 1), lambda i: (0, i, 0)),
          pl.BlockSpec((_H, 64), lambda i: (0, 0)),
          pl.BlockSpec((1, 64), lambda i: (0, 0)),
      ],
      out_specs=pl.BlockSpec((_BM, 64), lambda i: (i, 0)),
      out_shape=jax.ShapeDtypeStruct((_N, 64), jnp.float32),
  )(s, inv, w, b)


def _pad_adj(a):
  src = jnp.pad(a[0].reshape(_NS, _EPS), ((0, 0), (0, _EPP - _EPS)))
  dst = jnp.pad(a[1].reshape(_NS, _EPS), ((0, 0), (0, _EPP - _EPS)),
                constant_values=_N)
  return jnp.stack([src, dst]).reshape(2, _NS, _NBLK, _BLK, _K)


def kernel(x, adj1, adj2, W_fc1, b_fc1, W0, b0, W1, b1, W2, b2):
  adj = jnp.stack([_pad_adj(adj1), _pad_adj(adj2)])
  zeros = jnp.zeros((_CW, _H), jnp.float32)

  h = _fc1(x, W_fc1, b_fc1.reshape(1, _H))

  degs = _deg(jnp.ones((_K, _H), jnp.float32), adj, zeros)
  inv = 1.0 / jnp.clip(degs[:, :_N, :1], 1.0, None)

  s = _prop(h, adj, zeros)
  h = _layer(s, inv, h, W0, b0.reshape(1, _H))
  s = _prop(h, adj, zeros)
  h = _layer(s, inv, h, W1, b1.reshape(1, _H))

  w2p = jnp.pad(W2, ((0, 0), (0, 64 - _C)))
  b2p = jnp.pad(b2, (0, 64 - _C)).reshape(1, 64)
  s = _prop(h, adj, zeros)
  out = _layer2(s, inv, w2p, b2p)
  return out[:, :_C]
